# baseline (speedup 1.0000x reference)
import functools

import jax
import jax.numpy as jnp
from jax import lax
from jax.experimental import pallas as pl
from jax.experimental.pallas import tpu as pltpu
from jax.experimental.pallas import tpu_sc as plsc

_B = 16384
_L = 16
_NC = 2
_NS = 16
_NW = _NC * _NS
_BPW = _B // _NW


def _sc_pred_body(uid_hbm, iid_hbm, utab_hbm, atab_hbm, btab_hbm, pred_hbm,
                  uidx_v, iidx_v, urows_v, arows_v, b_v, pred_v, prod_v,
                  sem_u, sem_a, sem_b):
    wid = lax.axis_index("s") * _NC + lax.axis_index("c")
    base = wid * _BPW
    pltpu.sync_copy(uid_hbm.at[pl.ds(base, _BPW)], uidx_v)
    pltpu.sync_copy(iid_hbm.at[pl.ds(base, _BPW)], iidx_v)
    cu = pltpu.async_copy(utab_hbm.at[uidx_v], urows_v, sem_u)
    ca = pltpu.async_copy(atab_hbm.at[iidx_v], arows_v, sem_a)
    cb = pltpu.async_copy(btab_hbm.at[iidx_v], b_v, sem_b)
    cu.wait()
    ca.wait()
    cb.wait()

    lanes = lax.iota(jnp.int32, _L)

    def block(blk, _):
        base = blk * _L
        for j in range(_L):
            prod_v[pl.ds(j * _L, _L)] = urows_v[base + j] * arows_v[base + j]
        acc = jnp.zeros((_L,), jnp.float32)
        for c in range(_L):
            acc = acc + plsc.load_gather(prod_v, [lanes * _L + c])
        z = acc - b_v[pl.ds(base, _L)]
        pred_v[pl.ds(base, _L)] = 1.0 / (1.0 + jnp.exp(-z))
        return ()

    lax.fori_loop(0, _BPW // _L, block, ())
    pltpu.sync_copy(pred_v, pred_hbm.at[pl.ds(base, _BPW)])


_sc_pred = pl.kernel(
    _sc_pred_body,
    out_type=jax.ShapeDtypeStruct((_B,), jnp.float32),
    mesh=plsc.VectorSubcoreMesh(core_axis_name="c", subcore_axis_name="s"),
    compiler_params=pltpu.CompilerParams(
        needs_layout_passes=False, use_tc_tiling_on_sc=False),
    scratch_types=[
        pltpu.VMEM((_BPW,), jnp.int32),
        pltpu.VMEM((_BPW,), jnp.int32),
        pltpu.VMEM((_BPW, _L), jnp.float32),
        pltpu.VMEM((_BPW, _L), jnp.float32),
        pltpu.VMEM((_BPW,), jnp.float32),
        pltpu.VMEM((_BPW,), jnp.float32),
        pltpu.VMEM((_L * _L,), jnp.float32),
        pltpu.SemaphoreType.DMA,
        pltpu.SemaphoreType.DMA,
        pltpu.SemaphoreType.DMA,
    ],
)


def _loss_body(pred_ref, score_ref, loss_ref):
    eps = 1e-12
    p = jnp.clip(pred_ref[...], eps, 1.0 - eps)
    s = score_ref[...]
    t = s * jnp.log(p) + (1.0 - s) * jnp.log(1.0 - p)
    loss_ref[0, 0] = -jnp.sum(t) / _B


_loss_call = pl.pallas_call(
    _loss_body,
    out_shape=jax.ShapeDtypeStruct((1, 1), jnp.float32),
    out_specs=pl.BlockSpec(memory_space=pltpu.SMEM),
)


def kernel(user_id, item_id, score, user_table, a_table, b_table):
    b_flat = b_table.reshape(-1)
    pred = _sc_pred(user_id, item_id, user_table, a_table, b_flat)
    loss = _loss_call(pred.reshape(128, 128), score.reshape(128, 128))[0, 0]
    return pred, loss
